# Initial kernel scaffold; baseline (speedup 1.0000x reference)
import jax
import jax.numpy as jnp
from jax.experimental import pallas as pl


def kernel(x, neigh, weights):
    raise NotImplementedError("write your pallas kernel here")


# speedup vs baseline: 8.0278x; 8.0278x over previous
import functools

import jax
import jax.numpy as jnp
from jax import lax
from jax.experimental import pallas as pl
from jax.experimental.pallas import tpu as pltpu
from jax.experimental.pallas import tpu_sc as plsc

DIM_FILTER = 27
C = 16
NW = 32
GCH = 128
BN8 = 400


def _tc_transform(x8, wbd, npad8):
    nblk = npad8 // BN8

    def body(x_ref, w_ref, y_ref):
        k = pl.program_id(1)
        y_ref[...] = jnp.dot(x_ref[...], w_ref[k],
                             preferred_element_type=jnp.float32)

    return pl.pallas_call(
        body,
        grid=(nblk, DIM_FILTER),
        in_specs=[pl.BlockSpec((BN8, 128), lambda i, k: (i, 0)),
                  pl.BlockSpec((DIM_FILTER, 128, 128), lambda i, k: (0, 0, 0))],
        out_specs=pl.BlockSpec((BN8, 128), lambda i, k: (k * nblk + i, 0)),
        out_shape=jax.ShapeDtypeStruct((DIM_FILTER * npad8, 128),
                                       jnp.float32),
    )(x8, wbd)


def _sc_bag(npad, ch):
    ng = ch // GCH
    mesh = plsc.VectorSubcoreMesh(core_axis_name="c", subcore_axis_name="s")

    @functools.partial(
        pl.kernel,
        out_type=jax.ShapeDtypeStruct((npad, C), jnp.float32),
        mesh=mesh,
        scratch_types=[pltpu.VMEM((ch,), jnp.int32),
                       pltpu.VMEM((ch, C), jnp.float32),
                       pltpu.VMEM((ch, C), jnp.float32),
                       pltpu.SemaphoreType.DMA],
        compiler_params=pltpu.CompilerParams(use_tc_tiling_on_sc=False),
    )
    def kfn(yt, g_hbm, out_hbm, idx_v, rows_v, acc_v, sem):
        wid = lax.axis_index("s") * 2 + lax.axis_index("c")
        base = wid * ch

        def gather_into(dst_v):
            cps = [pltpu.async_copy(
                       yt.at[idx_v.at[pl.ds(g * GCH, GCH)]],
                       dst_v.at[pl.ds(g * GCH, GCH)], sem)
                   for g in range(ng)]
            for cp in cps:
                cp.wait()

        pltpu.sync_copy(g_hbm.at[pl.ds(base, ch)], idx_v)
        gather_into(acc_v)

        @pl.loop(1, DIM_FILTER)
        def _k(k):
            pltpu.sync_copy(g_hbm.at[pl.ds(k * npad + base, ch)], idx_v)
            gather_into(rows_v)

            @plsc.parallel_loop(0, ch, 1, unroll=8)
            def _r(r):
                plsc.addupdate(acc_v.at[r], rows_v[r])

        pltpu.sync_copy(acc_v, out_hbm.at[pl.ds(base, ch)])

    return kfn


def kernel(x, neigh, weights):
    n = x.shape[0]
    ch = ((n + NW - 1) // NW + GCH - 1) // GCH * GCH
    npad = ch * NW

    x32 = x.astype(jnp.float32)
    if npad > n:
        xp = jnp.concatenate(
            [x32, jnp.zeros((npad - n, C), jnp.float32)], axis=0)
    else:
        xp = x32
    x8 = xp.reshape(npad // 8, 128)

    eye8 = jnp.eye(8, dtype=jnp.float32)
    wbd = jax.vmap(lambda w: jnp.kron(eye8, w))(weights.astype(jnp.float32))

    y128 = _tc_transform(x8, wbd, npad // 8)
    yflat = y128.reshape(DIM_FILTER * npad, C)

    gi = (neigh.astype(jnp.int32).T
          + (jnp.arange(DIM_FILTER, dtype=jnp.int32) * npad)[:, None])
    gflat = jnp.pad(gi, ((0, 0), (0, npad - n))).reshape(-1)

    out = _sc_bag(npad, ch)(yflat, gflat)
    return out[:n]


# speedup vs baseline: 8.5648x; 1.0669x over previous
import functools

import jax
import jax.numpy as jnp
from jax import lax
from jax.experimental import pallas as pl
from jax.experimental.pallas import tpu as pltpu
from jax.experimental.pallas import tpu_sc as plsc

DIM_FILTER = 27
C = 16
NW = 32
GCH = 128
BN8 = 400


def _tc_transform(x8, wbd, npad8):
    nblk = npad8 // BN8

    def body(x_ref, w_ref, y_ref):
        k = pl.program_id(1)
        y_ref[...] = jnp.dot(x_ref[...], w_ref[k],
                             preferred_element_type=jnp.float32)

    return pl.pallas_call(
        body,
        grid=(nblk, DIM_FILTER),
        in_specs=[pl.BlockSpec((BN8, 128), lambda i, k: (i, 0)),
                  pl.BlockSpec((DIM_FILTER, 128, 128), lambda i, k: (0, 0, 0))],
        out_specs=pl.BlockSpec((BN8, 128), lambda i, k: (k * nblk + i, 0)),
        out_shape=jax.ShapeDtypeStruct((DIM_FILTER * npad8, 128),
                                       jnp.float32),
    )(x8, wbd)


def _sc_bag(npad, ch):
    ng = ch // GCH
    mesh = plsc.VectorSubcoreMesh(core_axis_name="c", subcore_axis_name="s")

    @functools.partial(
        pl.kernel,
        out_type=jax.ShapeDtypeStruct((npad, C), jnp.float32),
        mesh=mesh,
        scratch_types=[pltpu.VMEM((ch,), jnp.int32),
                       pltpu.VMEM((ch, C), jnp.float32),
                       pltpu.VMEM((ch, C), jnp.float32),
                       pltpu.SemaphoreType.DMA],
        compiler_params=pltpu.CompilerParams(use_tc_tiling_on_sc=False),
    )
    def kfn(yt, g_hbm, out_hbm, idx_v, rows_v, acc_v, sem):
        wid = lax.axis_index("s") * 2 + lax.axis_index("c")
        base = wid * ch

        def gather_into(dst_v, add):
            cps = [pltpu.async_copy(
                       yt.at[idx_v.at[pl.ds(g * GCH, GCH)]],
                       dst_v.at[pl.ds(g * GCH, GCH)], sem, add=add)
                   for g in range(ng)]
            for cp in cps:
                cp.wait()

        pltpu.sync_copy(g_hbm.at[pl.ds(base, ch)], idx_v)
        gather_into(acc_v, False)

        @pl.loop(1, DIM_FILTER)
        def _k(k):
            pltpu.sync_copy(g_hbm.at[pl.ds(k * npad + base, ch)], idx_v)
            gather_into(acc_v, True)

        pltpu.sync_copy(acc_v, out_hbm.at[pl.ds(base, ch)])

    return kfn


def kernel(x, neigh, weights):
    n = x.shape[0]
    ch = ((n + NW - 1) // NW + GCH - 1) // GCH * GCH
    npad = ch * NW

    x32 = x.astype(jnp.float32)
    if npad > n:
        xp = jnp.concatenate(
            [x32, jnp.zeros((npad - n, C), jnp.float32)], axis=0)
    else:
        xp = x32
    x8 = xp.reshape(npad // 8, 128)

    eye8 = jnp.eye(8, dtype=jnp.float32)
    wbd = jax.vmap(lambda w: jnp.kron(eye8, w))(weights.astype(jnp.float32))

    y128 = _tc_transform(x8, wbd, npad // 8)
    yflat = y128.reshape(DIM_FILTER * npad, C)

    gi = (neigh.astype(jnp.int32).T
          + (jnp.arange(DIM_FILTER, dtype=jnp.int32) * npad)[:, None])
    gflat = jnp.pad(gi, ((0, 0), (0, npad - n))).reshape(-1)

    out = _sc_bag(npad, ch)(yflat, gflat)
    return out[:n]


# speedup vs baseline: 11.5719x; 1.3511x over previous
import functools

import jax
import jax.numpy as jnp
from jax import lax
from jax.experimental import pallas as pl
from jax.experimental.pallas import tpu as pltpu
from jax.experimental.pallas import tpu_sc as plsc

DIM_FILTER = 27
C = 16
NW = 32
GCH = 128


def _tc_transform(x8, wbd, npad8):
    def body(x_ref, w_ref, y_ref):
        y_ref[...] = jnp.dot(x_ref[...], w_ref[0],
                             preferred_element_type=jnp.float32)

    return pl.pallas_call(
        body,
        grid=(DIM_FILTER,),
        in_specs=[pl.BlockSpec((npad8, 128), lambda k: (0, 0)),
                  pl.BlockSpec((1, 128, 128), lambda k: (k, 0, 0))],
        out_specs=pl.BlockSpec((npad8, 128), lambda k: (k, 0)),
        out_shape=jax.ShapeDtypeStruct((DIM_FILTER * npad8, 128),
                                       jnp.float32),
    )(x8, wbd)


def _sc_bag(npad, ch):
    ng = ch // GCH
    mesh = plsc.VectorSubcoreMesh(core_axis_name="c", subcore_axis_name="s")

    @functools.partial(
        pl.kernel,
        out_type=jax.ShapeDtypeStruct((npad, C), jnp.float32),
        mesh=mesh,
        scratch_types=[pltpu.VMEM((ch,), jnp.int32),
                       pltpu.VMEM((ch,), jnp.int32),
                       pltpu.VMEM((ch, C), jnp.float32),
                       pltpu.SemaphoreType.DMA,
                       pltpu.SemaphoreType.DMA],
        compiler_params=pltpu.CompilerParams(use_tc_tiling_on_sc=False),
    )
    def kfn(yt, g_hbm, out_hbm, idx_a, idx_b, acc_v, sem_a, sem_b):
        wid = lax.axis_index("s") * 2 + lax.axis_index("c")
        base = wid * ch
        bufs = (idx_a, idx_b)
        sems = (sem_a, sem_b)

        @plsc.parallel_loop(0, ch, 1, unroll=8)
        def _z(r):
            acc_v[r] = jnp.zeros((C,), jnp.float32)

        def fire(buf, sem):
            for g in range(ng):
                pltpu.async_copy(yt.at[buf.at[pl.ds(g * GCH, GCH)]],
                                 acc_v.at[pl.ds(g * GCH, GCH)], sem,
                                 add=True)

        def drain(buf, sem):
            for g in range(ng):
                pltpu.make_async_copy(yt.at[buf.at[pl.ds(g * GCH, GCH)]],
                                      acc_v.at[pl.ds(g * GCH, GCH)],
                                      sem).wait()

        for k in range(DIM_FILTER):
            par = k % 2
            if k >= 2:
                drain(bufs[par], sems[par])
            pltpu.sync_copy(g_hbm.at[pl.ds(k * npad + base, ch)], bufs[par])
            fire(bufs[par], sems[par])

        drain(bufs[(DIM_FILTER - 2) % 2], sems[(DIM_FILTER - 2) % 2])
        drain(bufs[(DIM_FILTER - 1) % 2], sems[(DIM_FILTER - 1) % 2])
        pltpu.sync_copy(acc_v, out_hbm.at[pl.ds(base, ch)])

    return kfn


def kernel(x, neigh, weights):
    n = x.shape[0]
    ch = ((n + NW - 1) // NW + GCH - 1) // GCH * GCH
    npad = ch * NW

    x32 = x.astype(jnp.float32)
    if npad > n:
        xp = jnp.concatenate(
            [x32, jnp.zeros((npad - n, C), jnp.float32)], axis=0)
    else:
        xp = x32
    x8 = xp.reshape(npad // 8, 128)

    eye8 = jnp.eye(8, dtype=jnp.float32)
    wbd = jax.vmap(lambda w: jnp.kron(eye8, w))(weights.astype(jnp.float32))

    y128 = _tc_transform(x8, wbd, npad // 8)
    yflat = y128.reshape(DIM_FILTER * npad, C)

    gi = (neigh.astype(jnp.int32).T
          + (jnp.arange(DIM_FILTER, dtype=jnp.int32) * npad)[:, None])
    gflat = jnp.pad(gi, ((0, 0), (0, npad - n))).reshape(-1)

    out = _sc_bag(npad, ch)(yflat, gflat)
    return out[:n]


# speedup vs baseline: 11.6964x; 1.0108x over previous
import functools

import jax
import jax.numpy as jnp
from jax import lax
from jax.experimental import pallas as pl
from jax.experimental.pallas import tpu as pltpu
from jax.experimental.pallas import tpu_sc as plsc

DIM_FILTER = 27
C = 16
NW = 32
GCH = 3200


def _tc_transform(x8, wbd, npad8):
    def body(x_ref, w_ref, y_ref):
        y_ref[...] = jnp.dot(x_ref[...], w_ref[0],
                             preferred_element_type=jnp.float32)

    return pl.pallas_call(
        body,
        grid=(DIM_FILTER,),
        in_specs=[pl.BlockSpec((npad8, 128), lambda k: (0, 0)),
                  pl.BlockSpec((1, 128, 128), lambda k: (k, 0, 0))],
        out_specs=pl.BlockSpec((npad8, 128), lambda k: (k, 0)),
        out_shape=jax.ShapeDtypeStruct((DIM_FILTER * npad8, 128),
                                       jnp.float32),
    )(x8, wbd)


def _sc_bag(npad, ch):
    ng = ch // GCH
    mesh = plsc.VectorSubcoreMesh(core_axis_name="c", subcore_axis_name="s")

    @functools.partial(
        pl.kernel,
        out_type=jax.ShapeDtypeStruct((npad, C), jnp.float32),
        mesh=mesh,
        scratch_types=[pltpu.VMEM((ch,), jnp.int32),
                       pltpu.VMEM((ch,), jnp.int32),
                       pltpu.VMEM((ch, C), jnp.float32),
                       pltpu.SemaphoreType.DMA,
                       pltpu.SemaphoreType.DMA],
        compiler_params=pltpu.CompilerParams(use_tc_tiling_on_sc=False),
    )
    def kfn(yt, g_hbm, out_hbm, idx_a, idx_b, acc_v, sem_a, sem_b):
        wid = lax.axis_index("s") * 2 + lax.axis_index("c")
        base = wid * ch
        bufs = (idx_a, idx_b)
        sems = (sem_a, sem_b)

        @plsc.parallel_loop(0, ch, 1, unroll=8)
        def _z(r):
            acc_v[r] = jnp.zeros((C,), jnp.float32)

        def fire(buf, sem):
            for g in range(ng):
                pltpu.async_copy(yt.at[buf.at[pl.ds(g * GCH, GCH)]],
                                 acc_v.at[pl.ds(g * GCH, GCH)], sem,
                                 add=True)

        def drain(buf, sem):
            for g in range(ng):
                pltpu.make_async_copy(yt.at[buf.at[pl.ds(g * GCH, GCH)]],
                                      acc_v.at[pl.ds(g * GCH, GCH)],
                                      sem).wait()

        for k in range(DIM_FILTER):
            par = k % 2
            if k >= 2:
                drain(bufs[par], sems[par])
            pltpu.sync_copy(g_hbm.at[pl.ds(k * npad + base, ch)], bufs[par])
            fire(bufs[par], sems[par])

        drain(bufs[(DIM_FILTER - 2) % 2], sems[(DIM_FILTER - 2) % 2])
        drain(bufs[(DIM_FILTER - 1) % 2], sems[(DIM_FILTER - 1) % 2])
        pltpu.sync_copy(acc_v, out_hbm.at[pl.ds(base, ch)])

    return kfn


def kernel(x, neigh, weights):
    n = x.shape[0]
    ch = ((n + NW - 1) // NW + GCH - 1) // GCH * GCH
    npad = ch * NW

    x32 = x.astype(jnp.float32)
    if npad > n:
        xp = jnp.concatenate(
            [x32, jnp.zeros((npad - n, C), jnp.float32)], axis=0)
    else:
        xp = x32
    x8 = xp.reshape(npad // 8, 128)

    eye8 = jnp.eye(8, dtype=jnp.float32)
    wbd = jax.vmap(lambda w: jnp.kron(eye8, w))(weights.astype(jnp.float32))

    y128 = _tc_transform(x8, wbd, npad // 8)
    yflat = y128.reshape(DIM_FILTER * npad, C)

    gi = (neigh.astype(jnp.int32).T
          + (jnp.arange(DIM_FILTER, dtype=jnp.int32) * npad)[:, None])
    gflat = jnp.pad(gi, ((0, 0), (0, npad - n))).reshape(-1)

    out = _sc_bag(npad, ch)(yflat, gflat)
    return out[:n]


# speedup vs baseline: 15.0952x; 1.2906x over previous
import functools

import jax
import jax.numpy as jnp
from jax import lax
from jax.experimental import pallas as pl
from jax.experimental.pallas import tpu as pltpu
from jax.experimental.pallas import tpu_sc as plsc

DIM_FILTER = 27
C = 16
NW = 32
ALIGN = 128


def _sc_gather(npad, ch):
    mesh = plsc.VectorSubcoreMesh(core_axis_name="c", subcore_axis_name="s")

    @functools.partial(
        pl.kernel,
        out_type=jax.ShapeDtypeStruct((DIM_FILTER * npad, C), jnp.bfloat16),
        mesh=mesh,
        scratch_types=[pltpu.VMEM_SHARED((npad, C), jnp.bfloat16),
                       pltpu.VMEM((ch,), jnp.int32),
                       pltpu.VMEM((ch,), jnp.int32),
                       pltpu.VMEM((ch, C), jnp.bfloat16),
                       pltpu.VMEM((ch, C), jnp.bfloat16),
                       pltpu.SemaphoreType.DMA,
                       pltpu.SemaphoreType.DMA,
                       pltpu.SemaphoreType.DMA],
        compiler_params=pltpu.CompilerParams(use_tc_tiling_on_sc=False),
    )
    def kfn(x_hbm, g_hbm, out_hbm, xs_sh, idx_a, idx_b, row_a, row_b,
            sem_g, sem_wa, sem_wb):
        sid = lax.axis_index("s")
        wid = sid * 2 + lax.axis_index("c")
        base = wid * ch
        bufs = (idx_a, idx_b)
        rows = (row_a, row_b)
        wsems = (sem_wa, sem_wb)

        @pl.when(sid == 0)
        def _load():
            pltpu.sync_copy(x_hbm, xs_sh)

        plsc.subcore_barrier()

        for k in range(DIM_FILTER):
            par = k % 2
            if k >= 2:
                pltpu.make_async_copy(
                    rows[par], out_hbm.at[pl.ds(base, ch)], wsems[par]).wait()
            pltpu.sync_copy(g_hbm.at[pl.ds(k * npad + base, ch)], bufs[par])
            pltpu.async_copy(xs_sh.at[bufs[par]], rows[par], sem_g).wait()
            pltpu.async_copy(rows[par],
                             out_hbm.at[pl.ds(k * npad + base, ch)],
                             wsems[par])

        for k in (DIM_FILTER - 2, DIM_FILTER - 1):
            par = k % 2
            pltpu.make_async_copy(
                rows[par], out_hbm.at[pl.ds(base, ch)], wsems[par]).wait()

    return kfn


def _tc_gemm(g8, wbd, npad8):
    BN8 = 6400
    nblk = npad8 // BN8

    def body(g_ref, w_ref, o_ref):
        k = pl.program_id(1)
        contrib = jnp.dot(g_ref[...], w_ref[0],
                          preferred_element_type=jnp.float32)

        @pl.when(k == 0)
        def _init():
            o_ref[...] = contrib

        @pl.when(k > 0)
        def _acc():
            o_ref[...] += contrib

    return pl.pallas_call(
        body,
        grid=(nblk, DIM_FILTER),
        in_specs=[pl.BlockSpec((BN8, 128), lambda i, k: (k * nblk + i, 0)),
                  pl.BlockSpec((1, 128, 128), lambda i, k: (k, 0, 0))],
        out_specs=pl.BlockSpec((BN8, 128), lambda i, k: (i, 0)),
        out_shape=jax.ShapeDtypeStruct((npad8, 128), jnp.float32),
    )(g8, wbd)


def kernel(x, neigh, weights):
    n = x.shape[0]
    ch = ((n + NW - 1) // NW + ALIGN - 1) // ALIGN * ALIGN
    npad = ch * NW

    xb = x.astype(jnp.bfloat16)
    if npad > n:
        xb = jnp.concatenate(
            [xb, jnp.zeros((npad - n, C), jnp.bfloat16)], axis=0)

    gi = neigh.astype(jnp.int32).T
    gflat = jnp.pad(gi, ((0, 0), (0, npad - n))).reshape(-1)

    g = _sc_gather(npad, ch)(xb, gflat)
    g8 = g.reshape(DIM_FILTER * npad * C // 128, 128)

    eye8 = jnp.eye(8, dtype=jnp.float32)
    wbd = jax.vmap(lambda w: jnp.kron(eye8, w))(
        weights.astype(jnp.float32)).astype(jnp.bfloat16)

    out8 = _tc_gemm(g8, wbd, npad * C // 128)
    return out8.reshape(npad, C)[:n]
